# baseline (speedup 1.0000x reference)
import functools

import jax
import jax.numpy as jnp
from jax import lax
from jax.experimental import pallas as pl
from jax.experimental.pallas import tpu as pltpu
from jax.experimental.pallas import tpu_sc as plsc

Q = 1024
N = 100000
D = 64
K = 32
BQ = 256
BR = 2048
N_PAD = 100352
L = 16
CHUNKS = N_PAD // (4 * L)


def _dist_body(q_ref, r_ref, qs_ref, rs_ref, o_ref):
    q = q_ref[...]
    r = r_ref[...]
    dot = lax.dot_general(q, r, (((1,), (1,)), ((), ())),
                          preferred_element_type=jnp.float32)
    o_ref[...] = qs_ref[...] + rs_ref[...] - 2.0 * dot


def _sqrt_body(x_ref, o_ref):
    o_ref[...] = jnp.sqrt(jnp.maximum(x_ref[...], 0.0))


def _merge16(a0d, a0i, a1d, a1i, d, idx):
    cd, ci = plsc.sort_key_val(d, idx)
    crd = lax.rev(cd, (0,))
    cri = lax.rev(ci, (0,))
    sel = crd < a1d
    u_d = jnp.where(sel, crd, a1d)
    u_i = jnp.where(sel, cri, a1i)
    ud, ui = plsc.sort_key_val(u_d, u_i)
    rd = lax.rev(ud, (0,))
    ri = lax.rev(ui, (0,))
    sel2 = a0d <= rd
    l_d = jnp.where(sel2, a0d, rd)
    l_i = jnp.where(sel2, a0i, ri)
    h_d = jnp.where(sel2, rd, a0d)
    h_i = jnp.where(sel2, ri, a0i)
    n0d, n0i = plsc.sort_key_val(l_d, l_i)
    n1d, n1i = plsc.sort_key_val(h_d, h_i)
    thr = n1d.at[jnp.full((L,), L - 1, dtype=jnp.int32)].get(
        mode="promise_in_bounds")
    return (n0d, n0i, n1d, n1i, thr)


def _noop():
    return None


def _gather_lanes(x, ids):
    return x.at[ids].get(mode="promise_in_bounds")


def _splat_last(x):
    return _gather_lanes(x, jnp.full((L,), L - 1, dtype=jnp.int32))


def _cummax16(x):
    iota = lax.iota(jnp.int32, L)
    for s in (1, 2, 4, 8):
        g = _gather_lanes(x, jnp.maximum(iota - s, 0))
        x = jnp.maximum(x, jnp.where(iota >= s, g, 0))
    return x


def _lex_fixup(od_v, oi_v):
    iota = lax.iota(jnp.int32, L)
    d0 = od_v[pl.ds(0, L)]
    d1 = od_v[pl.ds(L, L)]
    i0 = oi_v[pl.ds(0, L)]
    i1 = oi_v[pl.ds(L, L)]
    sh = jnp.maximum(iota - 1, 0)
    neq0 = d0 != _gather_lanes(d0, sh)
    neq1 = jnp.where(iota == 0, d1 != _splat_last(d0),
                     d1 != _gather_lanes(d1, sh))
    rid0 = _cummax16(jnp.where(neq0, iota, 0))
    rid1 = jnp.maximum(_cummax16(jnp.where(neq1, iota + L, 0)),
                       _splat_last(rid0))
    k0 = jnp.bitwise_or(jnp.left_shift(rid0, 17), i0)
    k1 = jnp.bitwise_or(jnp.left_shift(rid1, 17), i1)
    s0k, s0p = plsc.sort_key_val(k0, iota)
    s1k, s1p = plsc.sort_key_val(k1, iota + L)
    r1k = lax.rev(s1k, (0,))
    r1p = lax.rev(s1p, (0,))
    sel = s0k <= r1k
    lk = jnp.where(sel, s0k, r1k)
    lp = jnp.where(sel, s0p, r1p)
    hk = jnp.where(sel, r1k, s0k)
    hp = jnp.where(sel, r1p, s0p)
    _f0k, f0p = plsc.sort_key_val(lk, lp)
    _f1k, f1p = plsc.sort_key_val(hk, hp)
    nd0 = plsc.load_gather(od_v, [f0p])
    nd1 = plsc.load_gather(od_v, [f1p])
    ni0 = plsc.load_gather(oi_v, [f0p])
    ni1 = plsc.load_gather(oi_v, [f1p])
    od_v[pl.ds(0, L)] = nd0
    od_v[pl.ds(L, L)] = nd1
    oi_v[pl.ds(0, L)] = ni0
    oi_v[pl.ds(L, L)] = ni1


def _any_lanes(m):
    cnt = plsc.all_reduce_population_count(m)
    return cnt[0] != 0


HALF = N_PAD // 2
STRIPE = 32 * L
SB = STRIPE // (4 * L)
STRIPES_H = HALF // STRIPE


def _topk_kernel_body(dist_hbm, outd_hbm, outi_hbm, buf_a, buf_b, od_v, oi_v,
                      thr_v, sem_a, sem_b):
    info = plsc.get_sparse_core_info()
    nc = info.num_cores
    ns = info.num_subcores
    nw = nc * ns
    qn = Q // nw
    wid = lax.axis_index("s") * nc + lax.axis_index("c")

    def merge_chunk(buf, pbase, off):
        for t in range(4):
            def sub(t=t):
                dt = buf[pl.ds(pbase + t * L, L)]
                idx = off + pbase + t * L + lax.iota(jnp.int32, L)
                a0d = od_v[pl.ds(0, L)]
                a1d = od_v[pl.ds(L, L)]
                a0i = oi_v[pl.ds(0, L)]
                a1i = oi_v[pl.ds(L, L)]
                n0d, n0i, n1d, n1i, nthr = _merge16(
                    a0d, a0i, a1d, a1i, dt, idx)
                od_v[pl.ds(0, L)] = n0d
                od_v[pl.ds(L, L)] = n1d
                oi_v[pl.ds(0, L)] = n0i
                oi_v[pl.ds(L, L)] = n1i
                thr_v[pl.ds(0, L)] = nthr

            dt2 = buf[pl.ds(pbase + t * L, L)]
            thr2 = thr_v[pl.ds(0, L)]
            lax.cond(_any_lanes(dt2 < thr2), sub, _noop)

    def block64_scan(buf, b64, off):
        d0 = buf[pl.ds(b64, L)]
        d1 = buf[pl.ds(b64 + L, L)]
        d2 = buf[pl.ds(b64 + 2 * L, L)]
        d3 = buf[pl.ds(b64 + 3 * L, L)]
        thr = thr_v[pl.ds(0, L)]
        mn = jnp.minimum(jnp.minimum(d0, d1), jnp.minimum(d2, d3))
        lax.cond(_any_lanes(mn < thr), lambda: merge_chunk(buf, b64, off),
                 _noop)

    def process_stripe(buf, sbase, off):
        for t in range(SB):
            block64_scan(buf, sbase + t * (4 * L), off)

    def scan_half(buf, off):
        def stripe_iter(j, prev_hit):
            base = j * STRIPE
            thr = thr_v[pl.ds(0, L)]
            macc = None
            for t in range(SB):
                b = base + t * (4 * L)
                d0 = buf[pl.ds(b, L)]
                d1 = buf[pl.ds(b + L, L)]
                d2 = buf[pl.ds(b + 2 * L, L)]
                d3 = buf[pl.ds(b + 3 * L, L)]
                mn = jnp.minimum(jnp.minimum(d0, d1), jnp.minimum(d2, d3))
                m = mn < thr
                macc = m if t == 0 else macc | m
            hit = _any_lanes(macc)
            lax.cond(prev_hit,
                     lambda: process_stripe(buf, j * STRIPE - STRIPE, off),
                     _noop)
            return hit

        last = lax.fori_loop(0, STRIPES_H, stripe_iter, jnp.bool_(False))
        lax.cond(last, lambda: process_stripe(buf, HALF - STRIPE, off), _noop)

    def start_half(q, half, buf, sem):
        pltpu.async_copy(dist_hbm.at[q, pl.ds(half * HALF, HALF)], buf, sem)

    def wait_half(q, buf, sem):
        pltpu.make_async_copy(dist_hbm.at[q, pl.ds(0, HALF)], buf, sem).wait()

    q0 = wid * qn
    start_half(q0, 0, buf_a, sem_a)

    def per_query(qi, carry):
        q = wid * qn + qi
        inf = jnp.full((L,), jnp.inf, dtype=jnp.float32)
        zero = jnp.zeros((L,), dtype=jnp.int32)
        od_v[pl.ds(0, L)] = inf
        od_v[pl.ds(L, L)] = inf
        oi_v[pl.ds(0, L)] = zero
        oi_v[pl.ds(L, L)] = zero
        thr_v[pl.ds(0, L)] = inf

        wait_half(q, buf_a, sem_a)
        start_half(q, 1, buf_b, sem_b)
        scan_half(buf_a, 0)
        wait_half(q, buf_b, sem_b)

        @pl.when(qi < qn - 1)
        def _():
            start_half(q + 1, 0, buf_a, sem_a)

        scan_half(buf_b, HALF)
        _lex_fixup(od_v, oi_v)
        pltpu.sync_copy(od_v, outd_hbm.at[q])
        pltpu.sync_copy(oi_v, outi_hbm.at[q])
        return carry

    lax.fori_loop(0, qn, per_query, 0)


def kernel(queries, refs):
    refs_p = jnp.concatenate(
        [refs, jnp.zeros((N_PAD - N, D), dtype=refs.dtype)], axis=0)
    q_sq = jnp.sum(queries * queries, axis=1, keepdims=True)
    r_sq = jnp.sum(refs * refs, axis=1)
    r_sq_p = jnp.concatenate(
        [r_sq, jnp.full((N_PAD - N,), 3e8, dtype=r_sq.dtype)])[None, :]

    dist = pl.pallas_call(
        _dist_body,
        grid=(Q // BQ, N_PAD // BR),
        in_specs=[
            pl.BlockSpec((BQ, D), lambda i, j: (i, 0)),
            pl.BlockSpec((BR, D), lambda i, j: (j, 0)),
            pl.BlockSpec((BQ, 1), lambda i, j: (i, 0)),
            pl.BlockSpec((1, BR), lambda i, j: (0, j)),
        ],
        out_specs=pl.BlockSpec((BQ, BR), lambda i, j: (i, j)),
        out_shape=jax.ShapeDtypeStruct((Q, N_PAD), jnp.float32),
        compiler_params=pltpu.CompilerParams(
            dimension_semantics=("parallel", "arbitrary")),
    )(queries, refs_p, q_sq, r_sq_p)

    mesh = plsc.VectorSubcoreMesh(core_axis_name="c", subcore_axis_name="s")
    topk = functools.partial(
        pl.kernel,
        mesh=mesh,
        out_type=[
            jax.ShapeDtypeStruct((Q, K), jnp.float32),
            jax.ShapeDtypeStruct((Q, K), jnp.int32),
        ],
        scratch_types=[
            pltpu.VMEM((HALF,), jnp.float32),
            pltpu.VMEM((HALF,), jnp.float32),
            pltpu.VMEM((K,), jnp.float32),
            pltpu.VMEM((K,), jnp.int32),
            pltpu.VMEM((L,), jnp.float32),
            pltpu.SemaphoreType.DMA,
            pltpu.SemaphoreType.DMA,
        ],
        compiler_params=pltpu.CompilerParams(needs_layout_passes=False),
    )(_topk_kernel_body)

    knn_sq, knn_idx = topk(dist)

    knn_dist = pl.pallas_call(
        _sqrt_body,
        out_shape=jax.ShapeDtypeStruct((Q, K), jnp.float32),
    )(knn_sq)
    return knn_dist, knn_idx


# speedup vs baseline: 1.0147x; 1.0147x over previous
import functools

import jax
import jax.numpy as jnp
from jax import lax
from jax.experimental import pallas as pl
from jax.experimental.pallas import tpu as pltpu
from jax.experimental.pallas import tpu_sc as plsc

Q = 1024
N = 100000
D = 64
K = 32
BQ = 256
BR = 2048
N_PAD = 100352
L = 16
CHUNKS = N_PAD // (4 * L)


def _dist_body(q_ref, r_ref, qs_ref, rs_ref, o_ref):
    q = q_ref[...]
    r = r_ref[...]
    dot = lax.dot_general(q, r, (((1,), (1,)), ((), ())),
                          preferred_element_type=jnp.float32)
    o_ref[...] = qs_ref[...] + rs_ref[...] - 2.0 * dot


def _sqrt_body(x_ref, o_ref):
    o_ref[...] = jnp.sqrt(jnp.maximum(x_ref[...], 0.0))


def _merge16(a0d, a0i, a1d, a1i, d, idx):
    cd, ci = plsc.sort_key_val(d, idx)
    crd = lax.rev(cd, (0,))
    cri = lax.rev(ci, (0,))
    sel = crd < a1d
    u_d = jnp.where(sel, crd, a1d)
    u_i = jnp.where(sel, cri, a1i)
    ud, ui = plsc.sort_key_val(u_d, u_i)
    rd = lax.rev(ud, (0,))
    ri = lax.rev(ui, (0,))
    sel2 = a0d <= rd
    l_d = jnp.where(sel2, a0d, rd)
    l_i = jnp.where(sel2, a0i, ri)
    h_d = jnp.where(sel2, rd, a0d)
    h_i = jnp.where(sel2, ri, a0i)
    n0d, n0i = plsc.sort_key_val(l_d, l_i)
    n1d, n1i = plsc.sort_key_val(h_d, h_i)
    thr = n1d.at[jnp.full((L,), L - 1, dtype=jnp.int32)].get(
        mode="promise_in_bounds")
    return (n0d, n0i, n1d, n1i, thr)


def _noop():
    return None


def _gather_lanes(x, ids):
    return x.at[ids].get(mode="promise_in_bounds")


def _splat_last(x):
    return _gather_lanes(x, jnp.full((L,), L - 1, dtype=jnp.int32))


def _cummax16(x):
    iota = lax.iota(jnp.int32, L)
    for s in (1, 2, 4, 8):
        g = _gather_lanes(x, jnp.maximum(iota - s, 0))
        x = jnp.maximum(x, jnp.where(iota >= s, g, 0))
    return x


def _lex_fixup(od_v, oi_v):
    iota = lax.iota(jnp.int32, L)
    d0 = od_v[pl.ds(0, L)]
    d1 = od_v[pl.ds(L, L)]
    i0 = oi_v[pl.ds(0, L)]
    i1 = oi_v[pl.ds(L, L)]
    sh = jnp.maximum(iota - 1, 0)
    neq0 = d0 != _gather_lanes(d0, sh)
    neq1 = jnp.where(iota == 0, d1 != _splat_last(d0),
                     d1 != _gather_lanes(d1, sh))
    rid0 = _cummax16(jnp.where(neq0, iota, 0))
    rid1 = jnp.maximum(_cummax16(jnp.where(neq1, iota + L, 0)),
                       _splat_last(rid0))
    k0 = jnp.bitwise_or(jnp.left_shift(rid0, 17), i0)
    k1 = jnp.bitwise_or(jnp.left_shift(rid1, 17), i1)
    s0k, s0p = plsc.sort_key_val(k0, iota)
    s1k, s1p = plsc.sort_key_val(k1, iota + L)
    r1k = lax.rev(s1k, (0,))
    r1p = lax.rev(s1p, (0,))
    sel = s0k <= r1k
    lk = jnp.where(sel, s0k, r1k)
    lp = jnp.where(sel, s0p, r1p)
    hk = jnp.where(sel, r1k, s0k)
    hp = jnp.where(sel, r1p, s0p)
    _f0k, f0p = plsc.sort_key_val(lk, lp)
    _f1k, f1p = plsc.sort_key_val(hk, hp)
    nd0 = plsc.load_gather(od_v, [f0p])
    nd1 = plsc.load_gather(od_v, [f1p])
    ni0 = plsc.load_gather(oi_v, [f0p])
    ni1 = plsc.load_gather(oi_v, [f1p])
    od_v[pl.ds(0, L)] = nd0
    od_v[pl.ds(L, L)] = nd1
    oi_v[pl.ds(0, L)] = ni0
    oi_v[pl.ds(L, L)] = ni1


def _any_lanes(m):
    cnt = plsc.all_reduce_population_count(m)
    return cnt[0] != 0


HALF = N_PAD // 2
STRIPE = 16 * L
STRIPES_H = HALF // STRIPE


def _topk_kernel_body(dist_hbm, outd_hbm, outi_hbm, buf_a, buf_b, od_v, oi_v,
                      thr_v, sem_a, sem_b):
    info = plsc.get_sparse_core_info()
    nc = info.num_cores
    ns = info.num_subcores
    nw = nc * ns
    qn = Q // nw
    wid = lax.axis_index("s") * nc + lax.axis_index("c")

    def merge_chunk(buf, pbase, off):
        for t in range(4):
            def sub(t=t):
                dt = buf[pl.ds(pbase + t * L, L)]
                idx = off + pbase + t * L + lax.iota(jnp.int32, L)
                a0d = od_v[pl.ds(0, L)]
                a1d = od_v[pl.ds(L, L)]
                a0i = oi_v[pl.ds(0, L)]
                a1i = oi_v[pl.ds(L, L)]
                n0d, n0i, n1d, n1i, nthr = _merge16(
                    a0d, a0i, a1d, a1i, dt, idx)
                od_v[pl.ds(0, L)] = n0d
                od_v[pl.ds(L, L)] = n1d
                oi_v[pl.ds(0, L)] = n0i
                oi_v[pl.ds(L, L)] = n1i
                thr_v[pl.ds(0, L)] = nthr

            dt2 = buf[pl.ds(pbase + t * L, L)]
            thr2 = thr_v[pl.ds(0, L)]
            lax.cond(_any_lanes(dt2 < thr2), sub, _noop)

    def block64_scan(buf, b64, off):
        d0 = buf[pl.ds(b64, L)]
        d1 = buf[pl.ds(b64 + L, L)]
        d2 = buf[pl.ds(b64 + 2 * L, L)]
        d3 = buf[pl.ds(b64 + 3 * L, L)]
        thr = thr_v[pl.ds(0, L)]
        mn = jnp.minimum(jnp.minimum(d0, d1), jnp.minimum(d2, d3))
        lax.cond(_any_lanes(mn < thr), lambda: merge_chunk(buf, b64, off),
                 _noop)

    def process_stripe(buf, sbase, off):
        for t in range(4):
            block64_scan(buf, sbase + t * (4 * L), off)

    def scan_half(buf, off):
        def stripe_iter(j, prev_hit):
            base = j * STRIPE
            thr = thr_v[pl.ds(0, L)]
            macc = None
            for t in range(4):
                b = base + t * (4 * L)
                d0 = buf[pl.ds(b, L)]
                d1 = buf[pl.ds(b + L, L)]
                d2 = buf[pl.ds(b + 2 * L, L)]
                d3 = buf[pl.ds(b + 3 * L, L)]
                mn = jnp.minimum(jnp.minimum(d0, d1), jnp.minimum(d2, d3))
                m = mn < thr
                macc = m if t == 0 else macc | m
            hit = _any_lanes(macc)
            lax.cond(prev_hit,
                     lambda: process_stripe(buf, j * STRIPE - STRIPE, off),
                     _noop)
            return hit

        last = lax.fori_loop(0, STRIPES_H, stripe_iter, jnp.bool_(False))
        lax.cond(last, lambda: process_stripe(buf, HALF - STRIPE, off), _noop)

    def start_half(q, half, buf, sem):
        pltpu.async_copy(dist_hbm.at[q, pl.ds(half * HALF, HALF)], buf, sem)

    def wait_half(q, buf, sem):
        pltpu.make_async_copy(dist_hbm.at[q, pl.ds(0, HALF)], buf, sem).wait()

    q0 = wid * qn
    start_half(q0, 0, buf_a, sem_a)

    def per_query(qi, carry):
        q = wid * qn + qi
        inf = jnp.full((L,), jnp.inf, dtype=jnp.float32)
        zero = jnp.zeros((L,), dtype=jnp.int32)
        od_v[pl.ds(0, L)] = inf
        od_v[pl.ds(L, L)] = inf
        oi_v[pl.ds(0, L)] = zero
        oi_v[pl.ds(L, L)] = zero
        thr_v[pl.ds(0, L)] = inf

        wait_half(q, buf_a, sem_a)
        start_half(q, 1, buf_b, sem_b)
        scan_half(buf_a, 0)
        wait_half(q, buf_b, sem_b)

        @pl.when(qi < qn - 1)
        def _():
            start_half(q + 1, 0, buf_a, sem_a)

        scan_half(buf_b, HALF)
        _lex_fixup(od_v, oi_v)
        pltpu.sync_copy(od_v, outd_hbm.at[q])
        pltpu.sync_copy(oi_v, outi_hbm.at[q])
        return carry

    lax.fori_loop(0, qn, per_query, 0)


def kernel(queries, refs):
    refs_p = jnp.concatenate(
        [refs, jnp.zeros((N_PAD - N, D), dtype=refs.dtype)], axis=0)
    q_sq = jnp.sum(queries * queries, axis=1, keepdims=True)
    r_sq = jnp.sum(refs * refs, axis=1)
    r_sq_p = jnp.concatenate(
        [r_sq, jnp.full((N_PAD - N,), 3e8, dtype=r_sq.dtype)])[None, :]

    dist = pl.pallas_call(
        _dist_body,
        grid=(Q // BQ, N_PAD // BR),
        in_specs=[
            pl.BlockSpec((BQ, D), lambda i, j: (i, 0)),
            pl.BlockSpec((BR, D), lambda i, j: (j, 0)),
            pl.BlockSpec((BQ, 1), lambda i, j: (i, 0)),
            pl.BlockSpec((1, BR), lambda i, j: (0, j)),
        ],
        out_specs=pl.BlockSpec((BQ, BR), lambda i, j: (i, j)),
        out_shape=jax.ShapeDtypeStruct((Q, N_PAD), jnp.float32),
        compiler_params=pltpu.CompilerParams(
            dimension_semantics=("parallel", "arbitrary")),
    )(queries, refs_p, q_sq, r_sq_p)

    mesh = plsc.VectorSubcoreMesh(core_axis_name="c", subcore_axis_name="s")
    topk = functools.partial(
        pl.kernel,
        mesh=mesh,
        out_type=[
            jax.ShapeDtypeStruct((Q, K), jnp.float32),
            jax.ShapeDtypeStruct((Q, K), jnp.int32),
        ],
        scratch_types=[
            pltpu.VMEM((HALF,), jnp.float32),
            pltpu.VMEM((HALF,), jnp.float32),
            pltpu.VMEM((K,), jnp.float32),
            pltpu.VMEM((K,), jnp.int32),
            pltpu.VMEM((L,), jnp.float32),
            pltpu.SemaphoreType.DMA,
            pltpu.SemaphoreType.DMA,
        ],
        compiler_params=pltpu.CompilerParams(needs_layout_passes=False),
    )(_topk_kernel_body)

    knn_sq, knn_idx = topk(dist)

    knn_dist = pl.pallas_call(
        _sqrt_body,
        out_shape=jax.ShapeDtypeStruct((Q, K), jnp.float32),
    )(knn_sq)
    return knn_dist, knn_idx


# speedup vs baseline: 1.0681x; 1.0526x over previous
import functools

import jax
import jax.numpy as jnp
from jax import lax
from jax.experimental import pallas as pl
from jax.experimental.pallas import tpu as pltpu
from jax.experimental.pallas import tpu_sc as plsc

Q = 1024
N = 100000
D = 64
K = 32
BQ = 256
BR = 2048
N_PAD = 100352
L = 16
N_SLICES = 2


def _dist_body(q_ref, r_ref, qs_ref, rs_ref, o_ref):
    q = q_ref[...]
    r = r_ref[...]
    dot = lax.dot_general(q, r, (((1,), (1,)), ((), ())),
                          preferred_element_type=jnp.float32)
    o_ref[...] = qs_ref[...] + rs_ref[...] - 2.0 * dot


def _sqrt_body(x_ref, o_ref):
    o_ref[...] = jnp.sqrt(jnp.maximum(x_ref[...], 0.0))


def _merge16(a0d, a0i, a1d, a1i, d, idx):
    cd, ci = plsc.sort_key_val(d, idx)
    crd = lax.rev(cd, (0,))
    cri = lax.rev(ci, (0,))
    sel = crd < a1d
    u_d = jnp.where(sel, crd, a1d)
    u_i = jnp.where(sel, cri, a1i)
    ud, ui = plsc.sort_key_val(u_d, u_i)
    rd = lax.rev(ud, (0,))
    ri = lax.rev(ui, (0,))
    sel2 = a0d <= rd
    l_d = jnp.where(sel2, a0d, rd)
    l_i = jnp.where(sel2, a0i, ri)
    h_d = jnp.where(sel2, rd, a0d)
    h_i = jnp.where(sel2, ri, a0i)
    n0d, n0i = plsc.sort_key_val(l_d, l_i)
    n1d, n1i = plsc.sort_key_val(h_d, h_i)
    thr = n1d.at[jnp.full((L,), L - 1, dtype=jnp.int32)].get(
        mode="promise_in_bounds")
    return (n0d, n0i, n1d, n1i, thr)


def _noop():
    return None


def _gather_lanes(x, ids):
    return x.at[ids].get(mode="promise_in_bounds")


def _splat_last(x):
    return _gather_lanes(x, jnp.full((L,), L - 1, dtype=jnp.int32))


def _cummax16(x):
    iota = lax.iota(jnp.int32, L)
    for s in (1, 2, 4, 8):
        g = _gather_lanes(x, jnp.maximum(iota - s, 0))
        x = jnp.maximum(x, jnp.where(iota >= s, g, 0))
    return x


def _lex_fixup(od_v, oi_v):
    iota = lax.iota(jnp.int32, L)
    d0 = od_v[pl.ds(0, L)]
    d1 = od_v[pl.ds(L, L)]
    i0 = oi_v[pl.ds(0, L)]
    i1 = oi_v[pl.ds(L, L)]
    sh = jnp.maximum(iota - 1, 0)
    neq0 = d0 != _gather_lanes(d0, sh)
    neq1 = jnp.where(iota == 0, d1 != _splat_last(d0),
                     d1 != _gather_lanes(d1, sh))
    rid0 = _cummax16(jnp.where(neq0, iota, 0))
    rid1 = jnp.maximum(_cummax16(jnp.where(neq1, iota + L, 0)),
                       _splat_last(rid0))
    k0 = jnp.bitwise_or(jnp.left_shift(rid0, 17), i0)
    k1 = jnp.bitwise_or(jnp.left_shift(rid1, 17), i1)
    s0k, s0p = plsc.sort_key_val(k0, iota)
    s1k, s1p = plsc.sort_key_val(k1, iota + L)
    r1k = lax.rev(s1k, (0,))
    r1p = lax.rev(s1p, (0,))
    sel = s0k <= r1k
    lk = jnp.where(sel, s0k, r1k)
    lp = jnp.where(sel, s0p, r1p)
    hk = jnp.where(sel, r1k, s0k)
    hp = jnp.where(sel, r1p, s0p)
    _f0k, f0p = plsc.sort_key_val(lk, lp)
    _f1k, f1p = plsc.sort_key_val(hk, hp)
    nd0 = plsc.load_gather(od_v, [f0p])
    nd1 = plsc.load_gather(od_v, [f1p])
    ni0 = plsc.load_gather(oi_v, [f0p])
    ni1 = plsc.load_gather(oi_v, [f1p])
    od_v[pl.ds(0, L)] = nd0
    od_v[pl.ds(L, L)] = nd1
    oi_v[pl.ds(0, L)] = ni0
    oi_v[pl.ds(L, L)] = ni1


def _any_lanes(m):
    cnt = plsc.all_reduce_population_count(m)
    return cnt[0] != 0


HALF = N_PAD // 2
STRIPE = 16 * L
STRIPES_H = HALF // STRIPE


def _topk_kernel_body(dist_hbm, outd_hbm, outi_hbm, buf_a, buf_b, od_v, oi_v,
                      thr_v, sem_a, sem_b):
    info = plsc.get_sparse_core_info()
    nc = info.num_cores
    ns = info.num_subcores
    nw = nc * ns
    qn = dist_hbm.shape[0] // nw
    wid = lax.axis_index("s") * nc + lax.axis_index("c")

    def merge_chunk(buf, pbase, off):
        for t in range(4):
            def sub(t=t):
                dt = buf[pl.ds(pbase + t * L, L)]
                idx = off + pbase + t * L + lax.iota(jnp.int32, L)
                a0d = od_v[pl.ds(0, L)]
                a1d = od_v[pl.ds(L, L)]
                a0i = oi_v[pl.ds(0, L)]
                a1i = oi_v[pl.ds(L, L)]
                n0d, n0i, n1d, n1i, nthr = _merge16(
                    a0d, a0i, a1d, a1i, dt, idx)
                od_v[pl.ds(0, L)] = n0d
                od_v[pl.ds(L, L)] = n1d
                oi_v[pl.ds(0, L)] = n0i
                oi_v[pl.ds(L, L)] = n1i
                thr_v[pl.ds(0, L)] = nthr

            dt2 = buf[pl.ds(pbase + t * L, L)]
            thr2 = thr_v[pl.ds(0, L)]
            lax.cond(_any_lanes(dt2 < thr2), sub, _noop)

    def block64_scan(buf, b64, off):
        d0 = buf[pl.ds(b64, L)]
        d1 = buf[pl.ds(b64 + L, L)]
        d2 = buf[pl.ds(b64 + 2 * L, L)]
        d3 = buf[pl.ds(b64 + 3 * L, L)]
        thr = thr_v[pl.ds(0, L)]
        mn = jnp.minimum(jnp.minimum(d0, d1), jnp.minimum(d2, d3))
        lax.cond(_any_lanes(mn < thr), lambda: merge_chunk(buf, b64, off),
                 _noop)

    def process_stripe(buf, sbase, off):
        for t in range(4):
            block64_scan(buf, sbase + t * (4 * L), off)

    def scan_half(buf, off):
        def stripe_iter(j, prev_hit):
            base = j * STRIPE
            thr = thr_v[pl.ds(0, L)]
            macc = None
            for t in range(4):
                b = base + t * (4 * L)
                d0 = buf[pl.ds(b, L)]
                d1 = buf[pl.ds(b + L, L)]
                d2 = buf[pl.ds(b + 2 * L, L)]
                d3 = buf[pl.ds(b + 3 * L, L)]
                mn = jnp.minimum(jnp.minimum(d0, d1), jnp.minimum(d2, d3))
                m = mn < thr
                macc = m if t == 0 else macc | m
            hit = _any_lanes(macc)
            lax.cond(prev_hit,
                     lambda: process_stripe(buf, j * STRIPE - STRIPE, off),
                     _noop)
            return hit

        last = lax.fori_loop(0, STRIPES_H, stripe_iter, jnp.bool_(False))
        lax.cond(last, lambda: process_stripe(buf, HALF - STRIPE, off), _noop)

    def start_half(q, half, buf, sem):
        pltpu.async_copy(dist_hbm.at[q, pl.ds(half * HALF, HALF)], buf, sem)

    def wait_half(q, buf, sem):
        pltpu.make_async_copy(dist_hbm.at[q, pl.ds(0, HALF)], buf, sem).wait()

    q0 = wid * qn
    start_half(q0, 0, buf_a, sem_a)

    def per_query(qi, carry):
        q = wid * qn + qi
        inf = jnp.full((L,), jnp.inf, dtype=jnp.float32)
        zero = jnp.zeros((L,), dtype=jnp.int32)
        od_v[pl.ds(0, L)] = inf
        od_v[pl.ds(L, L)] = inf
        oi_v[pl.ds(0, L)] = zero
        oi_v[pl.ds(L, L)] = zero
        thr_v[pl.ds(0, L)] = inf

        wait_half(q, buf_a, sem_a)
        start_half(q, 1, buf_b, sem_b)
        scan_half(buf_a, 0)
        wait_half(q, buf_b, sem_b)

        @pl.when(qi < qn - 1)
        def _():
            start_half(q + 1, 0, buf_a, sem_a)

        scan_half(buf_b, HALF)
        _lex_fixup(od_v, oi_v)
        pltpu.sync_copy(od_v, outd_hbm.at[q])
        pltpu.sync_copy(oi_v, outi_hbm.at[q])
        return carry

    lax.fori_loop(0, qn, per_query, 0)


def kernel(queries, refs):
    refs_p = jnp.concatenate(
        [refs, jnp.zeros((N_PAD - N, D), dtype=refs.dtype)], axis=0)
    q_sq = jnp.sum(queries * queries, axis=1, keepdims=True)
    r_sq = jnp.sum(refs * refs, axis=1)
    r_sq_p = jnp.concatenate(
        [r_sq, jnp.full((N_PAD - N,), 3e8, dtype=r_sq.dtype)])[None, :]

    mesh = plsc.VectorSubcoreMesh(core_axis_name="c", subcore_axis_name="s")
    QS = Q // N_SLICES
    sq_parts, idx_parts = [], []
    for s in range(N_SLICES):
        dist = pl.pallas_call(
            _dist_body,
            grid=(QS // BQ, N_PAD // BR),
            in_specs=[
                pl.BlockSpec((BQ, D), lambda i, j: (i, 0)),
                pl.BlockSpec((BR, D), lambda i, j: (j, 0)),
                pl.BlockSpec((BQ, 1), lambda i, j: (i, 0)),
                pl.BlockSpec((1, BR), lambda i, j: (0, j)),
            ],
            out_specs=pl.BlockSpec((BQ, BR), lambda i, j: (i, j)),
            out_shape=jax.ShapeDtypeStruct((QS, N_PAD), jnp.float32),
            compiler_params=pltpu.CompilerParams(
                dimension_semantics=("parallel", "arbitrary")),
        )(queries[s * QS:(s + 1) * QS], refs_p,
          q_sq[s * QS:(s + 1) * QS], r_sq_p)

        topk = functools.partial(
            pl.kernel,
            mesh=mesh,
            out_type=[
                jax.ShapeDtypeStruct((QS, K), jnp.float32),
                jax.ShapeDtypeStruct((QS, K), jnp.int32),
            ],
            scratch_types=[
                pltpu.VMEM((HALF,), jnp.float32),
                pltpu.VMEM((HALF,), jnp.float32),
                pltpu.VMEM((K,), jnp.float32),
                pltpu.VMEM((K,), jnp.int32),
                pltpu.VMEM((L,), jnp.float32),
                pltpu.SemaphoreType.DMA,
                pltpu.SemaphoreType.DMA,
            ],
            compiler_params=pltpu.CompilerParams(needs_layout_passes=False),
        )(_topk_kernel_body)
        part_sq, part_idx = topk(dist)
        sq_parts.append(part_sq)
        idx_parts.append(part_idx)

    knn_sq = jnp.concatenate(sq_parts, axis=0)
    knn_idx = jnp.concatenate(idx_parts, axis=0)

    knn_dist = pl.pallas_call(
        _sqrt_body,
        out_shape=jax.ShapeDtypeStruct((Q, K), jnp.float32),
    )(knn_sq)
    return knn_dist, knn_idx


# speedup vs baseline: 1.0852x; 1.0160x over previous
import functools

import jax
import jax.numpy as jnp
from jax import lax
from jax.experimental import pallas as pl
from jax.experimental.pallas import tpu as pltpu
from jax.experimental.pallas import tpu_sc as plsc

Q = 1024
N = 100000
D = 64
K = 32
BQ = 256
BR = 2048
N_PAD = 100352
L = 16
N_SLICES = 4


def _dist_body(q_ref, r_ref, qs_ref, rs_ref, o_ref):
    q = q_ref[...]
    r = r_ref[...]
    dot = lax.dot_general(q, r, (((1,), (1,)), ((), ())),
                          preferred_element_type=jnp.float32)
    o_ref[...] = qs_ref[...] + rs_ref[...] - 2.0 * dot


def _sqrt_body(x_ref, o_ref):
    o_ref[...] = jnp.sqrt(jnp.maximum(x_ref[...], 0.0))


def _merge16(a0d, a0i, a1d, a1i, d, idx):
    cd, ci = plsc.sort_key_val(d, idx)
    crd = lax.rev(cd, (0,))
    cri = lax.rev(ci, (0,))
    sel = crd < a1d
    u_d = jnp.where(sel, crd, a1d)
    u_i = jnp.where(sel, cri, a1i)
    ud, ui = plsc.sort_key_val(u_d, u_i)
    rd = lax.rev(ud, (0,))
    ri = lax.rev(ui, (0,))
    sel2 = a0d <= rd
    l_d = jnp.where(sel2, a0d, rd)
    l_i = jnp.where(sel2, a0i, ri)
    h_d = jnp.where(sel2, rd, a0d)
    h_i = jnp.where(sel2, ri, a0i)
    n0d, n0i = plsc.sort_key_val(l_d, l_i)
    n1d, n1i = plsc.sort_key_val(h_d, h_i)
    thr = n1d.at[jnp.full((L,), L - 1, dtype=jnp.int32)].get(
        mode="promise_in_bounds")
    return (n0d, n0i, n1d, n1i, thr)


def _noop():
    return None


def _gather_lanes(x, ids):
    return x.at[ids].get(mode="promise_in_bounds")


def _splat_last(x):
    return _gather_lanes(x, jnp.full((L,), L - 1, dtype=jnp.int32))


def _cummax16(x):
    iota = lax.iota(jnp.int32, L)
    for s in (1, 2, 4, 8):
        g = _gather_lanes(x, jnp.maximum(iota - s, 0))
        x = jnp.maximum(x, jnp.where(iota >= s, g, 0))
    return x


def _lex_fixup(od_v, oi_v):
    iota = lax.iota(jnp.int32, L)
    d0 = od_v[pl.ds(0, L)]
    d1 = od_v[pl.ds(L, L)]
    i0 = oi_v[pl.ds(0, L)]
    i1 = oi_v[pl.ds(L, L)]
    sh = jnp.maximum(iota - 1, 0)
    neq0 = d0 != _gather_lanes(d0, sh)
    neq1 = jnp.where(iota == 0, d1 != _splat_last(d0),
                     d1 != _gather_lanes(d1, sh))
    rid0 = _cummax16(jnp.where(neq0, iota, 0))
    rid1 = jnp.maximum(_cummax16(jnp.where(neq1, iota + L, 0)),
                       _splat_last(rid0))
    k0 = jnp.bitwise_or(jnp.left_shift(rid0, 17), i0)
    k1 = jnp.bitwise_or(jnp.left_shift(rid1, 17), i1)
    s0k, s0p = plsc.sort_key_val(k0, iota)
    s1k, s1p = plsc.sort_key_val(k1, iota + L)
    r1k = lax.rev(s1k, (0,))
    r1p = lax.rev(s1p, (0,))
    sel = s0k <= r1k
    lk = jnp.where(sel, s0k, r1k)
    lp = jnp.where(sel, s0p, r1p)
    hk = jnp.where(sel, r1k, s0k)
    hp = jnp.where(sel, r1p, s0p)
    _f0k, f0p = plsc.sort_key_val(lk, lp)
    _f1k, f1p = plsc.sort_key_val(hk, hp)
    nd0 = plsc.load_gather(od_v, [f0p])
    nd1 = plsc.load_gather(od_v, [f1p])
    ni0 = plsc.load_gather(oi_v, [f0p])
    ni1 = plsc.load_gather(oi_v, [f1p])
    od_v[pl.ds(0, L)] = nd0
    od_v[pl.ds(L, L)] = nd1
    oi_v[pl.ds(0, L)] = ni0
    oi_v[pl.ds(L, L)] = ni1


def _any_lanes(m):
    cnt = plsc.all_reduce_population_count(m)
    return cnt[0] != 0


HALF = N_PAD // 2
STRIPE = 16 * L
STRIPES_H = HALF // STRIPE


def _topk_kernel_body(dist_hbm, outd_hbm, outi_hbm, buf_a, buf_b, od_v, oi_v,
                      thr_v, sem_a, sem_b):
    info = plsc.get_sparse_core_info()
    nc = info.num_cores
    ns = info.num_subcores
    nw = nc * ns
    qn = dist_hbm.shape[0] // nw
    wid = lax.axis_index("s") * nc + lax.axis_index("c")

    def merge_chunk(buf, pbase, off):
        for t in range(4):
            def sub(t=t):
                dt = buf[pl.ds(pbase + t * L, L)]
                idx = off + pbase + t * L + lax.iota(jnp.int32, L)
                a0d = od_v[pl.ds(0, L)]
                a1d = od_v[pl.ds(L, L)]
                a0i = oi_v[pl.ds(0, L)]
                a1i = oi_v[pl.ds(L, L)]
                n0d, n0i, n1d, n1i, nthr = _merge16(
                    a0d, a0i, a1d, a1i, dt, idx)
                od_v[pl.ds(0, L)] = n0d
                od_v[pl.ds(L, L)] = n1d
                oi_v[pl.ds(0, L)] = n0i
                oi_v[pl.ds(L, L)] = n1i
                thr_v[pl.ds(0, L)] = nthr

            dt2 = buf[pl.ds(pbase + t * L, L)]
            thr2 = thr_v[pl.ds(0, L)]
            lax.cond(_any_lanes(dt2 < thr2), sub, _noop)

    def block64_scan(buf, b64, off):
        d0 = buf[pl.ds(b64, L)]
        d1 = buf[pl.ds(b64 + L, L)]
        d2 = buf[pl.ds(b64 + 2 * L, L)]
        d3 = buf[pl.ds(b64 + 3 * L, L)]
        thr = thr_v[pl.ds(0, L)]
        mn = jnp.minimum(jnp.minimum(d0, d1), jnp.minimum(d2, d3))
        lax.cond(_any_lanes(mn < thr), lambda: merge_chunk(buf, b64, off),
                 _noop)

    def process_stripe(buf, sbase, off):
        for t in range(4):
            block64_scan(buf, sbase + t * (4 * L), off)

    def scan_half(buf, off):
        def stripe_iter(j, prev_hit):
            base = j * STRIPE
            thr = thr_v[pl.ds(0, L)]
            macc = None
            for t in range(4):
                b = base + t * (4 * L)
                d0 = buf[pl.ds(b, L)]
                d1 = buf[pl.ds(b + L, L)]
                d2 = buf[pl.ds(b + 2 * L, L)]
                d3 = buf[pl.ds(b + 3 * L, L)]
                mn = jnp.minimum(jnp.minimum(d0, d1), jnp.minimum(d2, d3))
                m = mn < thr
                macc = m if t == 0 else macc | m
            hit = _any_lanes(macc)
            lax.cond(prev_hit,
                     lambda: process_stripe(buf, j * STRIPE - STRIPE, off),
                     _noop)
            return hit

        last = lax.fori_loop(0, STRIPES_H, stripe_iter, jnp.bool_(False))
        lax.cond(last, lambda: process_stripe(buf, HALF - STRIPE, off), _noop)

    def start_half(q, half, buf, sem):
        pltpu.async_copy(dist_hbm.at[q, pl.ds(half * HALF, HALF)], buf, sem)

    def wait_half(q, buf, sem):
        pltpu.make_async_copy(dist_hbm.at[q, pl.ds(0, HALF)], buf, sem).wait()

    q0 = wid * qn
    start_half(q0, 0, buf_a, sem_a)

    def per_query(qi, carry):
        q = wid * qn + qi
        inf = jnp.full((L,), jnp.inf, dtype=jnp.float32)
        zero = jnp.zeros((L,), dtype=jnp.int32)
        od_v[pl.ds(0, L)] = inf
        od_v[pl.ds(L, L)] = inf
        oi_v[pl.ds(0, L)] = zero
        oi_v[pl.ds(L, L)] = zero
        thr_v[pl.ds(0, L)] = inf

        wait_half(q, buf_a, sem_a)
        start_half(q, 1, buf_b, sem_b)
        scan_half(buf_a, 0)
        wait_half(q, buf_b, sem_b)

        @pl.when(qi < qn - 1)
        def _():
            start_half(q + 1, 0, buf_a, sem_a)

        scan_half(buf_b, HALF)
        _lex_fixup(od_v, oi_v)
        pltpu.sync_copy(od_v, outd_hbm.at[q])
        pltpu.sync_copy(oi_v, outi_hbm.at[q])
        return carry

    lax.fori_loop(0, qn, per_query, 0)


def kernel(queries, refs):
    refs_p = jnp.concatenate(
        [refs, jnp.zeros((N_PAD - N, D), dtype=refs.dtype)], axis=0)
    q_sq = jnp.sum(queries * queries, axis=1, keepdims=True)
    r_sq = jnp.sum(refs * refs, axis=1)
    r_sq_p = jnp.concatenate(
        [r_sq, jnp.full((N_PAD - N,), 3e8, dtype=r_sq.dtype)])[None, :]

    mesh = plsc.VectorSubcoreMesh(core_axis_name="c", subcore_axis_name="s")
    QS = Q // N_SLICES
    sq_parts, idx_parts = [], []
    for s in range(N_SLICES):
        dist = pl.pallas_call(
            _dist_body,
            grid=(QS // BQ, N_PAD // BR),
            in_specs=[
                pl.BlockSpec((BQ, D), lambda i, j: (i, 0)),
                pl.BlockSpec((BR, D), lambda i, j: (j, 0)),
                pl.BlockSpec((BQ, 1), lambda i, j: (i, 0)),
                pl.BlockSpec((1, BR), lambda i, j: (0, j)),
            ],
            out_specs=pl.BlockSpec((BQ, BR), lambda i, j: (i, j)),
            out_shape=jax.ShapeDtypeStruct((QS, N_PAD), jnp.float32),
            compiler_params=pltpu.CompilerParams(
                dimension_semantics=("parallel", "arbitrary")),
        )(queries[s * QS:(s + 1) * QS], refs_p,
          q_sq[s * QS:(s + 1) * QS], r_sq_p)

        topk = functools.partial(
            pl.kernel,
            mesh=mesh,
            out_type=[
                jax.ShapeDtypeStruct((QS, K), jnp.float32),
                jax.ShapeDtypeStruct((QS, K), jnp.int32),
            ],
            scratch_types=[
                pltpu.VMEM((HALF,), jnp.float32),
                pltpu.VMEM((HALF,), jnp.float32),
                pltpu.VMEM((K,), jnp.float32),
                pltpu.VMEM((K,), jnp.int32),
                pltpu.VMEM((L,), jnp.float32),
                pltpu.SemaphoreType.DMA,
                pltpu.SemaphoreType.DMA,
            ],
            compiler_params=pltpu.CompilerParams(needs_layout_passes=False),
        )(_topk_kernel_body)
        part_sq, part_idx = topk(dist)
        sq_parts.append(part_sq)
        idx_parts.append(part_idx)

    knn_sq = jnp.concatenate(sq_parts, axis=0)
    knn_idx = jnp.concatenate(idx_parts, axis=0)

    knn_dist = pl.pallas_call(
        _sqrt_body,
        out_shape=jax.ShapeDtypeStruct((Q, K), jnp.float32),
    )(knn_sq)
    return knn_dist, knn_idx
